# baseline (device time: 28656 ns/iter reference)
import jax
import jax.numpy as jnp
from jax import lax
from jax.experimental import pallas as pl
from jax.experimental.pallas import tpu as pltpu

N_DEV = 4
N_EXPERTS = 16
EXPERTS_PER_DEV = N_EXPERTS // N_DEV
CAPACITY = 51
SLOTS = 64
SENT = 56
BLOCK = SLOTS * EXPERTS_PER_DEV


def kernel(x, router_W, route_idx, expert_W):
    n_tok, d_model = x.shape
    _, _, d_out = expert_W.shape

    def body(x_ref, route_ref, w_ref, out_ref, send_ref, comm_ref, send_sems, recv_sems):
        my = lax.axis_index("i")

        rvec = route_ref[:, 0]
        e_iota = lax.broadcasted_iota(jnp.int32, (n_tok, N_EXPERTS), 1)
        oh = (rvec[:, None] == e_iota).astype(jnp.bfloat16)
        ti = lax.broadcasted_iota(jnp.int32, (n_tok, n_tok), 0)
        tj = lax.broadcasted_iota(jnp.int32, (n_tok, n_tok), 1)
        L = (ti > tj).astype(jnp.bfloat16)
        prefix = jnp.dot(L, oh, preferred_element_type=jnp.float32)
        s = jnp.sum(prefix * oh.astype(jnp.float32), axis=1, keepdims=True)
        s = s.astype(jnp.int32)
        rows_col = jnp.where(s < CAPACITY, SLOTS * rvec[:, None] + s, -1)

        tok_iota = lax.broadcasted_iota(jnp.int32, (n_tok, BLOCK), 1)
        C_me = (tok_iota + BLOCK * my == rows_col).astype(jnp.bfloat16)

        comm_ref[:, :, :] = jnp.zeros((N_DEV, BLOCK, d_out), jnp.bfloat16)

        barrier_sem = pltpu.get_barrier_semaphore()
        for d in range(1, N_DEV):
            pl.semaphore_signal(
                barrier_sem,
                inc=1,
                device_id=((my + d) % N_DEV,),
                device_id_type=pl.DeviceIdType.MESH,
            )
        pl.semaphore_wait(barrier_sem, N_DEV - 1)

        x_bf = x_ref[:, :].astype(jnp.bfloat16)
        xc = lax.dot_general(
            C_me,
            x_bf,
            (((0,), (0,)), ((), ())),
            preferred_element_type=jnp.float32,
        )
        xc_bf = xc.astype(jnp.bfloat16)

        sends = []
        for j in range(EXPERTS_PER_DEV):
            sub = pl.ds(SLOTS * j, SLOTS)
            used = pl.ds(SLOTS * j, SENT)
            yj = jnp.dot(
                xc_bf[SLOTS * j : SLOTS * (j + 1), :],
                w_ref[j].astype(jnp.bfloat16),
                preferred_element_type=jnp.float32,
            )
            send_ref[sub, :] = yj.astype(jnp.bfloat16)
            for d in range(1, N_DEV):
                rdma = pltpu.make_async_remote_copy(
                    src_ref=send_ref.at[used],
                    dst_ref=comm_ref.at[my, used],
                    send_sem=send_sems.at[d - 1, j],
                    recv_sem=recv_sems.at[my, j],
                    device_id=((my + d) % N_DEV,),
                    device_id_type=pl.DeviceIdType.MESH,
                )
                rdma.start()
                sends.append(rdma)

        out_ref[:, :] = jnp.dot(
            C_me, send_ref[:, :], preferred_element_type=jnp.float32
        )

        for d in (1, N_DEV - 1, 2):
            k = (my + d) % N_DEV
            C_k = (tok_iota + BLOCK * k == rows_col).astype(jnp.bfloat16)
            for j in range(EXPERTS_PER_DEV):
                used = pl.ds(SLOTS * j, SENT)
                recv = pltpu.make_async_remote_copy(
                    src_ref=send_ref.at[used],
                    dst_ref=comm_ref.at[k, used],
                    send_sem=send_sems.at[0, j],
                    recv_sem=recv_sems.at[k, j],
                    device_id=(k,),
                    device_id_type=pl.DeviceIdType.MESH,
                )
                recv.wait_recv()
            out_ref[:, :] += jnp.dot(
                C_k, comm_ref[k], preferred_element_type=jnp.float32
            )

        for rdma in sends:
            rdma.wait_send()

    return pl.pallas_call(
        body,
        out_shape=jax.ShapeDtypeStruct((n_tok, d_out), jnp.float32),
        in_specs=[
            pl.BlockSpec(memory_space=pltpu.VMEM),
            pl.BlockSpec(memory_space=pltpu.VMEM),
            pl.BlockSpec(memory_space=pltpu.VMEM),
        ],
        out_specs=pl.BlockSpec(memory_space=pltpu.VMEM),
        scratch_shapes=[
            pltpu.VMEM((BLOCK, d_out), jnp.bfloat16),
            pltpu.VMEM((N_DEV, BLOCK, d_out), jnp.bfloat16),
            pltpu.SemaphoreType.DMA((N_DEV - 1, EXPERTS_PER_DEV)),
            pltpu.SemaphoreType.DMA((N_DEV, EXPERTS_PER_DEV)),
        ],
        compiler_params=pltpu.CompilerParams(collective_id=0),
    )(x, route_idx, expert_W)
